# Initial kernel scaffold; baseline (speedup 1.0000x reference)
import jax
import jax.numpy as jnp
from jax.experimental import pallas as pl


def kernel(Xl, Xg, Xe, C_vertex, C_edge, T_vertex, H_vertex, T_edge, H_edge, e_index, ling_w, ling_b, linl_w, linl_b, mha_wi, mha_bi, mha_wo, mha_bo, normg_g, normg_b, norml_w, norml_b, norml_ms, gat_w, gat_att_src, gat_att_dst, gat_b, lingS_w, lingS_b, linlS_w, linlS_b, lingD_w, linlD_w, mlp_w1, mlp_b1, mlp_lng, mlp_lnb, mlp_w2, mlp_b2):
    raise NotImplementedError("write your pallas kernel here")


# speedup vs baseline: 1.0128x; 1.0128x over previous
import functools

import jax
import jax.numpy as jnp
import numpy as np
from jax.experimental import pallas as pl
from jax.experimental.pallas import tpu as pltpu

NG = 2048
NL = 200000
NE = 20000
NT = 100000
NH = 100000
EL = 600000
DG = 128
DL = 128
H = 128
HEADS = 8
DH = H // HEADS
HID = 128
Q = 0.5


def _xg0_qkv_body(xg_ref, lw_ref, lb_ref, wi_ref, bi_ref, xg0_ref, qkv_ref):
    x = xg_ref[...]
    xg0 = jnp.tanh(
        jax.lax.dot_general(x, lw_ref[...], (((1,), (1,)), ((), ())))
        + lb_ref[...][None, :]
    )
    xg0_ref[...] = xg0
    qkv_ref[...] = (
        jax.lax.dot_general(xg0, wi_ref[...], (((1,), (1,)), ((), ())))
        + bi_ref[...][None, :]
    )


def _attn_body(q_ref, k_ref, v_ref, o_ref):
    q = q_ref[0]
    k = k_ref[0]
    v = v_ref[0]
    s = jax.lax.dot_general(q, k, (((1,), (1,)), ((), ()))) * (
        1.0 / np.sqrt(DH).astype(np.float32)
    )
    m = jnp.max(s, axis=1, keepdims=True)
    e = jnp.exp(s - m)
    p = e / jnp.sum(e, axis=1, keepdims=True)
    o_ref[0] = jnp.dot(p, v, preferred_element_type=jnp.float32)


def _xg_out_body(xg0_ref, at_ref, wo_ref, bo_ref, g_ref, b_ref, out_ref):
    xg0 = xg0_ref[...]
    xg1 = (
        jax.lax.dot_general(at_ref[...], wo_ref[...], (((1,), (1,)), ((), ())))
        + bo_ref[...][None, :]
    )
    y = xg0 + xg1
    mean = jnp.mean(y, axis=1, keepdims=True)
    var = jnp.mean((y - mean) ** 2, axis=1, keepdims=True)
    out_ref[...] = jnp.tanh(
        (y - mean) * jax.lax.rsqrt(var + 1e-5) * g_ref[...][None, :]
        + b_ref[...][None, :]
    )


def _xg_path(Xg, ling_w, ling_b, mha_wi, mha_bi, mha_wo, mha_bo, normg_g, normg_b):
    xg0, qkv = pl.pallas_call(
        _xg0_qkv_body,
        out_shape=(
            jax.ShapeDtypeStruct((NG, H), jnp.float32),
            jax.ShapeDtypeStruct((NG, 3 * H), jnp.float32),
        ),
    )(Xg, ling_w, ling_b, mha_wi, mha_bi)

    qkvh = qkv.reshape(NG, 3 * HEADS, DH).transpose(1, 0, 2)
    BQ = 256
    NQB = NG // BQ
    attnh = pl.pallas_call(
        _attn_body,
        grid=(HEADS, NQB),
        in_specs=[
            pl.BlockSpec((1, BQ, DH), lambda h, qb: (h, qb, 0)),
            pl.BlockSpec((1, NG, DH), lambda h, qb: (HEADS + h, 0, 0)),
            pl.BlockSpec((1, NG, DH), lambda h, qb: (2 * HEADS + h, 0, 0)),
        ],
        out_specs=pl.BlockSpec((1, BQ, DH), lambda h, qb: (h, qb, 0)),
        out_shape=jax.ShapeDtypeStruct((HEADS, NG, DH), jnp.float32),
    )(qkvh, qkvh, qkvh)
    attn = attnh.transpose(1, 0, 2).reshape(NG, H)

    xg2 = pl.pallas_call(
        _xg_out_body,
        out_shape=jax.ShapeDtypeStruct((NG, H), jnp.float32),
    )(xg0, attn, mha_wo, mha_bo, normg_g, normg_b)
    return xg2


def _linear_tanh_body(x_ref, w_ref, b_ref, o_ref):
    o_ref[...] = jnp.tanh(
        jax.lax.dot_general(x_ref[...], w_ref[...], (((1,), (1,)), ((), ())))
        + b_ref[...][None, :]
    )


def _xl_linear(Xl, linl_w, linl_b):
    BR = 2000
    return pl.pallas_call(
        _linear_tanh_body,
        grid=(NL // BR,),
        in_specs=[
            pl.BlockSpec((BR, DL), lambda i: (i, 0)),
            pl.BlockSpec((H, DL), lambda i: (0, 0)),
            pl.BlockSpec((H,), lambda i: (0,)),
        ],
        out_specs=pl.BlockSpec((BR, H), lambda i: (i, 0)),
        out_shape=jax.ShapeDtypeStruct((NL, H), jnp.float32),
    )(Xl, linl_w, linl_b)


def _mlp_body(x_ref, w1_ref, b1_ref, g_ref, b_ref, w2_ref, b2_ref, o_ref):
    hm = jax.nn.relu(
        jax.lax.dot_general(x_ref[...], w1_ref[...], (((1,), (1,)), ((), ())))
        + b1_ref[...][None, :]
    )
    mean = jnp.mean(hm, axis=1, keepdims=True)
    var = jnp.mean((hm - mean) ** 2, axis=1, keepdims=True)
    hm = (hm - mean) * jax.lax.rsqrt(var + 1e-5) * g_ref[...][None, :] + b_ref[...][
        None, :
    ]
    s = jnp.sum(hm * w2_ref[...], axis=1, keepdims=True) + b2_ref[0, 0]
    o_ref[...] = jax.nn.sigmoid(s)


def _edge_mlp(Xe, w1, b1, lng, lnb, w2, b2):
    BR = 2000
    return pl.pallas_call(
        _mlp_body,
        grid=(NE // BR,),
        in_specs=[
            pl.BlockSpec((BR, DG), lambda i: (i, 0)),
            pl.BlockSpec((HID, DG), lambda i: (0, 0)),
            pl.BlockSpec((HID,), lambda i: (0,)),
            pl.BlockSpec((HID,), lambda i: (0,)),
            pl.BlockSpec((HID,), lambda i: (0,)),
            pl.BlockSpec((1, HID), lambda i: (0, 0)),
            pl.BlockSpec(memory_space=pltpu.SMEM),
        ],
        out_specs=pl.BlockSpec((BR, 1), lambda i: (i, 0)),
        out_shape=jax.ShapeDtypeStruct((NE, 1), jnp.float32),
    )(Xe, w1, b1, lng, lnb, w2, b2.reshape(1, 1))


def _seg_sums_sq(vals, ids, num):
    s = jax.ops.segment_sum(vals, ids, num_segments=num)
    s2 = jax.ops.segment_sum(vals * vals, ids, num_segments=num)
    c = jax.ops.segment_sum(jnp.ones((vals.shape[0], 1), vals.dtype), ids,
                            num_segments=num)
    return s, s2, c


def _score(T_vals, H_vals, T_edge, H_edge, Sw, Sb, Dw):
    sT, sT2, cT = _seg_sums_sq(T_vals, T_edge, NE)
    sH, sH2, cH = _seg_sums_sq(H_vals, H_edge, NE)
    icT = 1.0 / jnp.maximum(cT, 1.0)
    icH = 1.0 / jnp.maximum(cH, 1.0)
    MT = sT * icT
    MT2 = sT2 * icT
    MH = sH * icH
    MH2 = sH2 * icH
    mskT = (cT > 0.0).astype(jnp.float32)
    mskH = (cH > 0.0).astype(jnp.float32)
    Diff_T = MT2 - MT * MT
    Diff_H = MH2 - MH * MH
    Diff_TH = (MT2 - 2.0 * MT * MH + MH * MH) * mskT
    Diff_HT = (MH2 - 2.0 * MH * MT + MT * MT) * mskH
    feats = jnp.concatenate([Diff_T, Diff_H, Diff_TH, Diff_HT], axis=1)
    Se = jax.nn.sigmoid(feats @ Sw.T + Sb)
    a = MT @ Dw.T
    b = MH
    na = jnp.maximum(jnp.linalg.norm(a, axis=1), 1e-8)
    nb = jnp.maximum(jnp.linalg.norm(b, axis=1), 1e-8)
    De = ((jnp.sum(a * b, axis=1) / (na * nb))[:, None] + 1.0) / 2.0
    return Se, De


def _group_norm(xl, C_edge, w, b, ms):
    s, s2, c = _seg_sums_sq(xl, C_edge, NE)
    ic = 1.0 / jnp.maximum(c, 1.0)
    mean = s * ic
    msq = s2 * ic
    cvec = mean * ms
    var = msq - 2.0 * cvec * mean + cvec * cvec
    mean_r = mean[C_edge]
    var_r = var[C_edge]
    return w * (xl - mean_r * ms) / jnp.sqrt(var_r + 1e-5) + b


def _gat(x, ei, W, att_src, att_dst, bias):
    h = (x @ W.T).reshape(NL, HEADS, DH)
    src = ei[0]
    dst = ei[1]
    a_src = jnp.sum(h * att_src[None, :, :], axis=-1)
    a_dst = jnp.sum(h * att_dst[None, :, :], axis=-1)
    alpha = jax.nn.leaky_relu(a_src[src] + a_dst[dst], negative_slope=0.2)
    amax = jax.ops.segment_max(alpha, dst, num_segments=NL)
    amax = jnp.where(jnp.isfinite(amax), amax, 0.0)
    ex = jnp.exp(alpha - amax[dst])
    denom = jax.ops.segment_sum(ex, dst, num_segments=NL)
    coef = ex / jnp.maximum(denom[dst], 1e-16)
    msg = h[src] * coef[:, :, None]
    out = jax.ops.segment_sum(msg, dst, num_segments=NL)
    return out.reshape(NL, H) + bias


def kernel(Xl, Xg, Xe, C_vertex, C_edge, T_vertex, H_vertex, T_edge, H_edge,
           e_index, ling_w, ling_b, linl_w, linl_b, mha_wi, mha_bi, mha_wo,
           mha_bo, normg_g, normg_b, norml_w, norml_b, norml_ms, gat_w,
           gat_att_src, gat_att_dst, gat_b, lingS_w, lingS_b, linlS_w,
           linlS_b, lingD_w, linlD_w, mlp_w1, mlp_b1, mlp_lng, mlp_lnb,
           mlp_w2, mlp_b2):
    xg2 = _xg_path(Xg, ling_w, ling_b, mha_wi, mha_bi, mha_wo, mha_bo,
                   normg_g, normg_b)

    CT = C_vertex[T_vertex]
    CH = C_vertex[H_vertex]
    Seg, Deg = _score(xg2[CT], xg2[CH], T_edge, H_edge, lingS_w, lingS_b,
                      lingD_w)

    xl1 = _xl_linear(Xl, linl_w, linl_b)
    xln = _group_norm(xl1, C_edge, norml_w, norml_b, norml_ms)
    xl2 = jnp.tanh(xln + _gat(xln, e_index, gat_w, gat_att_src, gat_att_dst,
                              gat_b))
    Sel, Del = _score(xl2[T_vertex], xl2[H_vertex], T_edge, H_edge, linlS_w,
                      linlS_b, linlD_w)

    Se = Q * Seg + (1.0 - Q) * Sel
    De = Q * Deg + (1.0 - Q) * Del

    Pe = _edge_mlp(Xe, mlp_w1, mlp_b1, mlp_lng, mlp_lnb, mlp_w2, mlp_b2)
    return Pe, Se, De


# speedup vs baseline: 13.2545x; 13.0864x over previous
import functools

import jax
import jax.numpy as jnp
import numpy as np
from jax import lax
from jax.experimental import pallas as pl
from jax.experimental.pallas import tpu as pltpu
from jax.experimental.pallas import tpu_sc as plsc

NG = 2048
NL = 200000
NE = 20000
NT = 100000
NH = 100000
EL = 600000
DG = 128
DL = 128
H = 128
HEADS = 8
DH = H // HEADS
HID = 128
Q = 0.5

_NC = 2
_NS = 16
_NW = _NC * _NS


def _sc_gather_call(table, idx, chunk):
    V, W = table.shape
    B = idx.shape[0]
    per_w = B // _NW
    n_chunks = per_w // chunk
    mesh = plsc.VectorSubcoreMesh(core_axis_name="c", subcore_axis_name="s")

    @functools.partial(
        pl.kernel,
        mesh=mesh,
        out_type=jax.ShapeDtypeStruct((B, W), jnp.float32),
        scratch_types=[
            pltpu.VMEM((chunk,), jnp.int32),
            pltpu.VMEM((chunk, W), jnp.float32),
            pltpu.SemaphoreType.DMA,
        ],
    )
    def k(table_hbm, idx_hbm, out_hbm, idx_v, rows_v, sem):
        wid = lax.axis_index("s") * _NC + lax.axis_index("c")
        base = wid * per_w

        def body(ci, _):
            off = base + ci * chunk
            pltpu.sync_copy(idx_hbm.at[pl.ds(off, chunk)], idx_v)
            pltpu.async_copy(table_hbm.at[idx_v], rows_v, sem).wait()
            pltpu.sync_copy(rows_v, out_hbm.at[pl.ds(off, chunk)])
            return ()

        lax.fori_loop(0, n_chunks, body, ())

    return k(table, idx)


def _pad_idx(idx, granule):
    n = idx.shape[0]
    m = ((n + granule - 1) // granule) * granule
    if m == n:
        return idx, n
    return jnp.concatenate([idx, jnp.zeros((m - n,), jnp.int32)]), n


def _sc_gather(table, idx):
    W = table.shape[1]
    chunk = 256 if W > 128 else 512
    pidx, n = _pad_idx(idx, chunk * _NW)
    out = _sc_gather_call(table, pidx, chunk)
    return out[:n]


def _xg0_qkv_body(xg_ref, lw_ref, lb_ref, wi_ref, bi_ref, xg0_ref, qkv_ref):
    x = xg_ref[...]
    xg0 = jnp.tanh(
        jax.lax.dot_general(x, lw_ref[...], (((1,), (1,)), ((), ())))
        + lb_ref[...][None, :]
    )
    xg0_ref[...] = xg0
    qkv_ref[...] = (
        jax.lax.dot_general(xg0, wi_ref[...], (((1,), (1,)), ((), ())))
        + bi_ref[...][None, :]
    )


def _attn_body(q_ref, k_ref, v_ref, o_ref):
    q = q_ref[0]
    k = k_ref[0]
    v = v_ref[0]
    s = jax.lax.dot_general(q, k, (((1,), (1,)), ((), ()))) * (
        1.0 / np.sqrt(DH).astype(np.float32)
    )
    m = jnp.max(s, axis=1, keepdims=True)
    e = jnp.exp(s - m)
    p = e / jnp.sum(e, axis=1, keepdims=True)
    o_ref[0] = jnp.dot(p, v, preferred_element_type=jnp.float32)


def _xg_out_body(xg0_ref, at_ref, wo_ref, bo_ref, g_ref, b_ref, out_ref):
    xg0 = xg0_ref[...]
    xg1 = (
        jax.lax.dot_general(at_ref[...], wo_ref[...], (((1,), (1,)), ((), ())))
        + bo_ref[...][None, :]
    )
    y = xg0 + xg1
    mean = jnp.mean(y, axis=1, keepdims=True)
    var = jnp.mean((y - mean) ** 2, axis=1, keepdims=True)
    xg2 = jnp.tanh(
        (y - mean) * jax.lax.rsqrt(var + 1e-5) * g_ref[...][None, :]
        + b_ref[...][None, :]
    )
    out_ref[...] = jnp.concatenate([xg2, xg2 * xg2], axis=1)


def _xg_path(Xg, ling_w, ling_b, mha_wi, mha_bi, mha_wo, mha_bo, normg_g, normg_b):
    xg0, qkv = pl.pallas_call(
        _xg0_qkv_body,
        out_shape=(
            jax.ShapeDtypeStruct((NG, H), jnp.float32),
            jax.ShapeDtypeStruct((NG, 3 * H), jnp.float32),
        ),
    )(Xg, ling_w, ling_b, mha_wi, mha_bi)

    qkvh = qkv.reshape(NG, 3 * HEADS, DH).transpose(1, 0, 2)
    BQ = 256
    NQB = NG // BQ
    attnh = pl.pallas_call(
        _attn_body,
        grid=(HEADS, NQB),
        in_specs=[
            pl.BlockSpec((1, BQ, DH), lambda h, qb: (h, qb, 0)),
            pl.BlockSpec((1, NG, DH), lambda h, qb: (HEADS + h, 0, 0)),
            pl.BlockSpec((1, NG, DH), lambda h, qb: (2 * HEADS + h, 0, 0)),
        ],
        out_specs=pl.BlockSpec((1, BQ, DH), lambda h, qb: (h, qb, 0)),
        out_shape=jax.ShapeDtypeStruct((HEADS, NG, DH), jnp.float32),
    )(qkvh, qkvh, qkvh)
    attn = attnh.transpose(1, 0, 2).reshape(NG, H)

    xg2pack = pl.pallas_call(
        _xg_out_body,
        out_shape=jax.ShapeDtypeStruct((NG, 2 * H), jnp.float32),
    )(xg0, attn, mha_wo, mha_bo, normg_g, normg_b)
    return xg2pack


def _linear_tanh_sq_body(x_ref, w_ref, b_ref, o_ref):
    y = jnp.tanh(
        jax.lax.dot_general(x_ref[...], w_ref[...], (((1,), (1,)), ((), ())))
        + b_ref[...][None, :]
    )
    o_ref[...] = jnp.concatenate([y, y * y], axis=1)


def _xl_linear(Xl, linl_w, linl_b):
    BR = 2000
    return pl.pallas_call(
        _linear_tanh_sq_body,
        grid=(NL // BR,),
        in_specs=[
            pl.BlockSpec((BR, DL), lambda i: (i, 0)),
            pl.BlockSpec((H, DL), lambda i: (0, 0)),
            pl.BlockSpec((H,), lambda i: (0,)),
        ],
        out_specs=pl.BlockSpec((BR, 2 * H), lambda i: (i, 0)),
        out_shape=jax.ShapeDtypeStruct((NL, 2 * H), jnp.float32),
    )(Xl, linl_w, linl_b)


def _gn_stats_body(s_ref, c_ref, w_ref, b_ref, ms_ref, ab_ref):
    c = jnp.maximum(c_ref[...], 1.0)
    ic = 1.0 / c
    mean = s_ref[:, :H] * ic
    msq = s_ref[:, H:] * ic
    cvec = mean * ms_ref[...][None, :]
    var = msq - 2.0 * cvec * mean + cvec * cvec
    A = w_ref[...][None, :] * jax.lax.rsqrt(var + 1e-5)
    B = b_ref[...][None, :] - cvec * A
    ab_ref[...] = jnp.concatenate([A, B], axis=1)


def _gn_stats(sums, counts, w, b, ms):
    BR = 2000
    return pl.pallas_call(
        _gn_stats_body,
        grid=(NE // BR,),
        in_specs=[
            pl.BlockSpec((BR, 2 * H), lambda i: (i, 0)),
            pl.BlockSpec((BR, 1), lambda i: (i, 0)),
            pl.BlockSpec((H,), lambda i: (0,)),
            pl.BlockSpec((H,), lambda i: (0,)),
            pl.BlockSpec((H,), lambda i: (0,)),
        ],
        out_specs=pl.BlockSpec((BR, 2 * H), lambda i: (i, 0)),
        out_shape=jax.ShapeDtypeStruct((NE, 2 * H), jnp.float32),
    )(sums, counts, w, b, ms)


def _gat_prep_body(x_ref, ab_ref, wg_ref, ms_ref, md_ref, xln_ref, hp_ref,
                   ad_ref, mxs_ref, mxd_ref):
    i = pl.program_id(0)
    xln = x_ref[:, :H] * ab_ref[:, :H] + ab_ref[:, H:]
    xln_ref[...] = xln
    h = jax.lax.dot_general(xln, wg_ref[...], (((1,), (1,)), ((), ())))
    asrcE = jnp.dot(h, ms_ref[...], preferred_element_type=jnp.float32)
    adstE = jnp.dot(h, md_ref[...], preferred_element_type=jnp.float32)
    hp_ref[...] = jnp.concatenate([h, asrcE], axis=1)
    ad_ref[...] = adstE
    bs = jnp.max(asrcE)
    bd = jnp.max(adstE)

    @pl.when(i == 0)
    def _():
        mxs_ref[0, 0] = bs
        mxd_ref[0, 0] = bd

    @pl.when(i > 0)
    def _():
        mxs_ref[0, 0] = jnp.maximum(mxs_ref[0, 0], bs)
        mxd_ref[0, 0] = jnp.maximum(mxd_ref[0, 0], bd)


def _gat_prep(xl1pack, ab_r, gat_w, M_s, M_d):
    BR = 2000
    return pl.pallas_call(
        _gat_prep_body,
        grid=(NL // BR,),
        in_specs=[
            pl.BlockSpec((BR, 2 * H), lambda i: (i, 0)),
            pl.BlockSpec((BR, 2 * H), lambda i: (i, 0)),
            pl.BlockSpec((H, H), lambda i: (0, 0)),
            pl.BlockSpec((H, H), lambda i: (0, 0)),
            pl.BlockSpec((H, H), lambda i: (0, 0)),
        ],
        out_specs=(
            pl.BlockSpec((BR, H), lambda i: (i, 0)),
            pl.BlockSpec((BR, 2 * H), lambda i: (i, 0)),
            pl.BlockSpec((BR, H), lambda i: (i, 0)),
            pl.BlockSpec(memory_space=pltpu.SMEM),
            pl.BlockSpec(memory_space=pltpu.SMEM),
        ),
        out_shape=(
            jax.ShapeDtypeStruct((NL, H), jnp.float32),
            jax.ShapeDtypeStruct((NL, 2 * H), jnp.float32),
            jax.ShapeDtypeStruct((NL, H), jnp.float32),
            jax.ShapeDtypeStruct((1, 1), jnp.float32),
            jax.ShapeDtypeStruct((1, 1), jnp.float32),
        ),
    )(xl1pack, ab_r, gat_w, M_s, M_d)


def _gat_edge_body(gs_ref, gd_ref, c_ref, o_ref):
    s = gs_ref[:, H:] + gd_ref[...]
    alpha = jnp.where(s >= 0.0, s, 0.2 * s)
    ex = jnp.exp(alpha - c_ref[0, 0])
    o_ref[...] = jnp.concatenate([gs_ref[:, :H] * ex, ex], axis=1)


def _gat_edge(g_src, g_dst, cmax):
    BR = 4000
    NEL = g_src.shape[0]
    return pl.pallas_call(
        _gat_edge_body,
        grid=(NEL // BR,),
        in_specs=[
            pl.BlockSpec((BR, 2 * H), lambda i: (i, 0)),
            pl.BlockSpec((BR, H), lambda i: (i, 0)),
            pl.BlockSpec(memory_space=pltpu.SMEM),
        ],
        out_specs=pl.BlockSpec((BR, 2 * H), lambda i: (i, 0)),
        out_shape=jax.ShapeDtypeStruct((NEL, 2 * H), jnp.float32),
    )(g_src, g_dst, cmax)


def _gat_fin_body(nd_ref, xln_ref, b_ref, o_ref):
    y = nd_ref[:, :H] / jnp.maximum(nd_ref[:, H:], 1e-16)
    xl2 = jnp.tanh(xln_ref[...] + y + b_ref[...][None, :])
    o_ref[...] = jnp.concatenate([xl2, xl2 * xl2], axis=1)


def _gat_fin(nd, xln, gat_b):
    BR = 2000
    return pl.pallas_call(
        _gat_fin_body,
        grid=(NL // BR,),
        in_specs=[
            pl.BlockSpec((BR, 2 * H), lambda i: (i, 0)),
            pl.BlockSpec((BR, H), lambda i: (i, 0)),
            pl.BlockSpec((H,), lambda i: (0,)),
        ],
        out_specs=pl.BlockSpec((BR, 2 * H), lambda i: (i, 0)),
        out_shape=jax.ShapeDtypeStruct((NL, 2 * H), jnp.float32),
    )(nd, xln, gat_b)


def _score_tail_body(gT_ref, gH_ref, lT_ref, lH_ref, cT_ref, cH_ref,
                     gSw_ref, lSw_ref, gDw_ref, lDw_ref, gSb_ref, lSb_ref,
                     se_ref, de_ref):
    icT = 1.0 / jnp.maximum(cT_ref[...], 1.0)
    icH = 1.0 / jnp.maximum(cH_ref[...], 1.0)
    mskT = (cT_ref[...] > 0.0).astype(jnp.float32)
    mskH = (cH_ref[...] > 0.0).astype(jnp.float32)

    def one(sums_T, sums_H, Sw, Sb, Dw):
        MT = sums_T[:, :H] * icT
        MT2 = sums_T[:, H:] * icT
        MH = sums_H[:, :H] * icH
        MH2 = sums_H[:, H:] * icH
        d_T = MT2 - MT * MT
        d_H = MH2 - MH * MH
        d_TH = (MT2 - 2.0 * MT * MH + MH * MH) * mskT
        d_HT = (MH2 - 2.0 * MH * MT + MT * MT) * mskH
        lin = (
            jnp.sum(d_T * Sw[0, :H][None, :], axis=1, keepdims=True)
            + jnp.sum(d_H * Sw[0, H:2 * H][None, :], axis=1, keepdims=True)
            + jnp.sum(d_TH * Sw[0, 2 * H:3 * H][None, :], axis=1, keepdims=True)
            + jnp.sum(d_HT * Sw[0, 3 * H:][None, :], axis=1, keepdims=True)
        )
        se = jax.nn.sigmoid(lin + Sb)
        a = jax.lax.dot_general(MT, Dw, (((1,), (1,)), ((), ())))
        na = jnp.maximum(
            jnp.sqrt(jnp.sum(a * a, axis=1, keepdims=True)), 1e-8)
        nb = jnp.maximum(
            jnp.sqrt(jnp.sum(MH * MH, axis=1, keepdims=True)), 1e-8)
        de = (jnp.sum(a * MH, axis=1, keepdims=True) / (na * nb) + 1.0) / 2.0
        return se, de

    se_g, de_g = one(gT_ref[...], gH_ref[...], gSw_ref[...], gSb_ref[0, 0],
                     gDw_ref[...])
    se_l, de_l = one(lT_ref[...], lH_ref[...], lSw_ref[...], lSb_ref[0, 0],
                     lDw_ref[...])
    se_ref[...] = Q * se_g + (1.0 - Q) * se_l
    de_ref[...] = Q * de_g + (1.0 - Q) * de_l


def _score_tail(sums_gT, sums_gH, sums_lT, sums_lH, cT, cH,
                gSw, gSb, lSw, lSb, gDw, lDw):
    BR = 2000
    sspec = pl.BlockSpec((BR, 2 * H), lambda i: (i, 0))
    cspec = pl.BlockSpec((BR, 1), lambda i: (i, 0))
    wspec = pl.BlockSpec((1, 4 * H), lambda i: (0, 0))
    dspec = pl.BlockSpec((H, H), lambda i: (0, 0))
    return pl.pallas_call(
        _score_tail_body,
        grid=(NE // BR,),
        in_specs=[sspec, sspec, sspec, sspec, cspec, cspec,
                  wspec, wspec, dspec, dspec,
                  pl.BlockSpec(memory_space=pltpu.SMEM),
                  pl.BlockSpec(memory_space=pltpu.SMEM)],
        out_specs=(cspec, cspec),
        out_shape=(
            jax.ShapeDtypeStruct((NE, 1), jnp.float32),
            jax.ShapeDtypeStruct((NE, 1), jnp.float32),
        ),
    )(sums_gT, sums_gH, sums_lT, sums_lH, cT, cH, gSw, lSw, gDw, lDw,
      gSb.reshape(1, 1), lSb.reshape(1, 1))


def _mlp_body(x_ref, w1_ref, b1_ref, g_ref, b_ref, w2_ref, b2_ref, o_ref):
    hm = jax.nn.relu(
        jax.lax.dot_general(x_ref[...], w1_ref[...], (((1,), (1,)), ((), ())))
        + b1_ref[...][None, :]
    )
    mean = jnp.mean(hm, axis=1, keepdims=True)
    var = jnp.mean((hm - mean) ** 2, axis=1, keepdims=True)
    hm = (hm - mean) * jax.lax.rsqrt(var + 1e-5) * g_ref[...][None, :] + b_ref[...][
        None, :
    ]
    s = jnp.sum(hm * w2_ref[...], axis=1, keepdims=True) + b2_ref[0, 0]
    o_ref[...] = jax.nn.sigmoid(s)


def _edge_mlp(Xe, w1, b1, lng, lnb, w2, b2):
    BR = 2000
    return pl.pallas_call(
        _mlp_body,
        grid=(NE // BR,),
        in_specs=[
            pl.BlockSpec((BR, DG), lambda i: (i, 0)),
            pl.BlockSpec((HID, DG), lambda i: (0, 0)),
            pl.BlockSpec((HID,), lambda i: (0,)),
            pl.BlockSpec((HID,), lambda i: (0,)),
            pl.BlockSpec((HID,), lambda i: (0,)),
            pl.BlockSpec((1, HID), lambda i: (0, 0)),
            pl.BlockSpec(memory_space=pltpu.SMEM),
        ],
        out_specs=pl.BlockSpec((BR, 1), lambda i: (i, 0)),
        out_shape=jax.ShapeDtypeStruct((NE, 1), jnp.float32),
    )(Xe, w1, b1, lng, lnb, w2, b2.reshape(1, 1))


def _seg_sum(vals, ids, num):
    return jax.ops.segment_sum(vals, ids, num_segments=num)


def kernel(Xl, Xg, Xe, C_vertex, C_edge, T_vertex, H_vertex, T_edge, H_edge,
           e_index, ling_w, ling_b, linl_w, linl_b, mha_wi, mha_bi, mha_wo,
           mha_bo, normg_g, normg_b, norml_w, norml_b, norml_ms, gat_w,
           gat_att_src, gat_att_dst, gat_b, lingS_w, lingS_b, linlS_w,
           linlS_b, lingD_w, linlD_w, mlp_w1, mlp_b1, mlp_lng, mlp_lnb,
           mlp_w2, mlp_b2):
    f32 = jnp.float32

    xg2pack = _xg_path(Xg, ling_w, ling_b, mha_wi, mha_bi, mha_wo, mha_bo,
                       normg_g, normg_b)

    CT = C_vertex[T_vertex]
    CH = C_vertex[H_vertex]
    gvT = _sc_gather(xg2pack, CT)
    gvH = _sc_gather(xg2pack, CH)
    cT = _seg_sum(jnp.ones((NT, 1), f32), T_edge, NE)
    cH = _seg_sum(jnp.ones((NH, 1), f32), H_edge, NE)
    sums_gT = _seg_sum(gvT, T_edge, NE)
    sums_gH = _seg_sum(gvH, H_edge, NE)

    xl1pack = _xl_linear(Xl, linl_w, linl_b)
    gn_sums = _seg_sum(xl1pack, C_edge, NE)
    gn_cnt = _seg_sum(jnp.ones((NL, 1), f32), C_edge, NE)
    ab = _gn_stats(gn_sums, gn_cnt, norml_w, norml_b, norml_ms)
    ab_r = _sc_gather(ab, C_edge)

    lane = jnp.arange(H, dtype=jnp.int32)
    head = lane // DH
    onehot = (head[:, None] == jnp.arange(HEADS, dtype=jnp.int32)[None, :]
              ).astype(f32)
    expand = (jnp.arange(HEADS, dtype=jnp.int32)[:, None] == head[None, :]
              ).astype(f32)
    M_s = (gat_att_src.reshape(H)[:, None] * onehot) @ expand
    M_d = (gat_att_dst.reshape(H)[:, None] * onehot) @ expand

    xln, hpack, adstE, mxs, mxd = _gat_prep(xl1pack, ab_r, gat_w, M_s, M_d)
    amax = mxs[0, 0] + mxd[0, 0]
    cmax = jnp.where(amax >= 0.0, amax, 0.2 * amax).reshape(1, 1)

    src = e_index[0]
    dst = e_index[1]
    g_src = _sc_gather(hpack, src)
    g_dst = _sc_gather(adstE, dst)
    edge_out = _gat_edge(g_src, g_dst, cmax)
    nd = _seg_sum(edge_out, dst, NL)
    xl2pack = _gat_fin(nd, xln, gat_b)

    lvT = _sc_gather(xl2pack, T_vertex)
    lvH = _sc_gather(xl2pack, H_vertex)
    sums_lT = _seg_sum(lvT, T_edge, NE)
    sums_lH = _seg_sum(lvH, H_edge, NE)

    Se, De = _score_tail(sums_gT, sums_gH, sums_lT, sums_lH, cT, cH,
                         lingS_w, lingS_b, linlS_w, linlS_b, lingD_w, linlD_w)

    Pe = _edge_mlp(Xe, mlp_w1, mlp_b1, mlp_lng, mlp_lnb, mlp_w2, mlp_b2)
    return Pe, Se, De


# speedup vs baseline: 13.8135x; 1.0422x over previous
import functools

import jax
import jax.numpy as jnp
import numpy as np
from jax import lax
from jax.experimental import pallas as pl
from jax.experimental.pallas import tpu as pltpu
from jax.experimental.pallas import tpu_sc as plsc

NG = 2048
NL = 200000
NE = 20000
NT = 100000
NH = 100000
EL = 600000
DG = 128
DL = 128
H = 128
HEADS = 8
DH = H // HEADS
HID = 128
Q = 0.5

_NC = 2
_NS = 16
_NW = _NC * _NS


def _sc_gather_call(table, idx, chunk):
    V, W = table.shape
    B = idx.shape[0]
    per_w = B // _NW
    n_chunks = per_w // chunk
    mesh = plsc.VectorSubcoreMesh(core_axis_name="c", subcore_axis_name="s")

    @functools.partial(
        pl.kernel,
        mesh=mesh,
        out_type=jax.ShapeDtypeStruct((B, W), jnp.float32),
        scratch_types=[
            pltpu.VMEM((chunk,), jnp.int32),
            pltpu.VMEM((chunk, W), jnp.float32),
            pltpu.SemaphoreType.DMA,
        ],
    )
    def k(table_hbm, idx_hbm, out_hbm, idx_v, rows_v, sem):
        wid = lax.axis_index("s") * _NC + lax.axis_index("c")
        base = wid * per_w

        def body(ci, _):
            off = base + ci * chunk
            pltpu.sync_copy(idx_hbm.at[pl.ds(off, chunk)], idx_v)
            pltpu.async_copy(table_hbm.at[idx_v], rows_v, sem).wait()
            pltpu.sync_copy(rows_v, out_hbm.at[pl.ds(off, chunk)])
            return ()

        lax.fori_loop(0, n_chunks, body, ())

    return k(table, idx)


def _pad_idx(idx, granule):
    n = idx.shape[0]
    m = ((n + granule - 1) // granule) * granule
    if m == n:
        return idx, n
    return jnp.concatenate([idx, jnp.zeros((m - n,), jnp.int32)]), n


def _sc_gather(table, idx):
    W = table.shape[1]
    if W <= 16:
        chunk = 1024
    elif W <= 128:
        chunk = 512
    elif W <= 256:
        chunk = 256
    else:
        chunk = 224
    pidx, n = _pad_idx(idx, chunk * _NW)
    out = _sc_gather_call(table, pidx, chunk)
    return out[:n]


def _xg0_qkv_body(xg_ref, lw_ref, lb_ref, wi_ref, bi_ref, xg0_ref, qkv_ref):
    x = xg_ref[...]
    xg0 = jnp.tanh(
        jax.lax.dot_general(x, lw_ref[...], (((1,), (1,)), ((), ())))
        + lb_ref[...][None, :]
    )
    xg0_ref[...] = xg0
    qkv_ref[...] = (
        jax.lax.dot_general(xg0, wi_ref[...], (((1,), (1,)), ((), ())))
        + bi_ref[...][None, :]
    )


def _attn_body(q_ref, k_ref, v_ref, o_ref):
    q = q_ref[0]
    k = k_ref[0]
    v = v_ref[0]
    s = jax.lax.dot_general(q, k, (((1,), (1,)), ((), ()))) * (
        1.0 / np.sqrt(DH).astype(np.float32)
    )
    m = jnp.max(s, axis=1, keepdims=True)
    e = jnp.exp(s - m)
    p = e / jnp.sum(e, axis=1, keepdims=True)
    o_ref[0] = jnp.dot(p, v, preferred_element_type=jnp.float32)


def _xg_out_body(xg0_ref, at_ref, wo_ref, bo_ref, g_ref, b_ref, out_ref):
    xg0 = xg0_ref[...]
    xg1 = (
        jax.lax.dot_general(at_ref[...], wo_ref[...], (((1,), (1,)), ((), ())))
        + bo_ref[...][None, :]
    )
    y = xg0 + xg1
    mean = jnp.mean(y, axis=1, keepdims=True)
    var = jnp.mean((y - mean) ** 2, axis=1, keepdims=True)
    xg2 = jnp.tanh(
        (y - mean) * jax.lax.rsqrt(var + 1e-5) * g_ref[...][None, :]
        + b_ref[...][None, :]
    )
    out_ref[...] = jnp.concatenate([xg2, xg2 * xg2], axis=1)


def _xg_path(Xg, ling_w, ling_b, mha_wi, mha_bi, mha_wo, mha_bo, normg_g, normg_b):
    xg0, qkv = pl.pallas_call(
        _xg0_qkv_body,
        out_shape=(
            jax.ShapeDtypeStruct((NG, H), jnp.float32),
            jax.ShapeDtypeStruct((NG, 3 * H), jnp.float32),
        ),
    )(Xg, ling_w, ling_b, mha_wi, mha_bi)

    qkvh = qkv.reshape(NG, 3 * HEADS, DH).transpose(1, 0, 2)
    BQ = 256
    NQB = NG // BQ
    attnh = pl.pallas_call(
        _attn_body,
        grid=(HEADS, NQB),
        in_specs=[
            pl.BlockSpec((1, BQ, DH), lambda h, qb: (h, qb, 0)),
            pl.BlockSpec((1, NG, DH), lambda h, qb: (HEADS + h, 0, 0)),
            pl.BlockSpec((1, NG, DH), lambda h, qb: (2 * HEADS + h, 0, 0)),
        ],
        out_specs=pl.BlockSpec((1, BQ, DH), lambda h, qb: (h, qb, 0)),
        out_shape=jax.ShapeDtypeStruct((HEADS, NG, DH), jnp.float32),
    )(qkvh, qkvh, qkvh)
    attn = attnh.transpose(1, 0, 2).reshape(NG, H)

    xg2pack = pl.pallas_call(
        _xg_out_body,
        out_shape=jax.ShapeDtypeStruct((NG, 2 * H), jnp.float32),
    )(xg0, attn, mha_wo, mha_bo, normg_g, normg_b)
    return xg2pack


def _linear_tanh_sq_body(x_ref, w_ref, b_ref, o_ref):
    y = jnp.tanh(
        jax.lax.dot_general(x_ref[...], w_ref[...], (((1,), (1,)), ((), ())))
        + b_ref[...][None, :]
    )
    ones = jnp.ones((y.shape[0], 8), jnp.float32)
    o_ref[...] = jnp.concatenate([y, y * y, ones], axis=1)


def _xl_linear(Xl, linl_w, linl_b):
    BR = 2000
    return pl.pallas_call(
        _linear_tanh_sq_body,
        grid=(NL // BR,),
        in_specs=[
            pl.BlockSpec((BR, DL), lambda i: (i, 0)),
            pl.BlockSpec((H, DL), lambda i: (0, 0)),
            pl.BlockSpec((H,), lambda i: (0,)),
        ],
        out_specs=pl.BlockSpec((BR, 2 * H + 8), lambda i: (i, 0)),
        out_shape=jax.ShapeDtypeStruct((NL, 2 * H + 8), jnp.float32),
    )(Xl, linl_w, linl_b)


def _gn_stats_body(s_ref, w_ref, b_ref, ms_ref, ab_ref):
    c = jnp.maximum(s_ref[:, 2 * H:2 * H + 1], 1.0)
    ic = 1.0 / c
    mean = s_ref[:, :H] * ic
    msq = s_ref[:, H:2 * H] * ic
    cvec = mean * ms_ref[...][None, :]
    var = msq - 2.0 * cvec * mean + cvec * cvec
    A = w_ref[...][None, :] * jax.lax.rsqrt(var + 1e-5)
    B = b_ref[...][None, :] - cvec * A
    ab_ref[...] = jnp.concatenate([A, B], axis=1)


def _gn_stats(sums, w, b, ms):
    BR = 2000
    return pl.pallas_call(
        _gn_stats_body,
        grid=(NE // BR,),
        in_specs=[
            pl.BlockSpec((BR, 2 * H + 8), lambda i: (i, 0)),
            pl.BlockSpec((H,), lambda i: (0,)),
            pl.BlockSpec((H,), lambda i: (0,)),
            pl.BlockSpec((H,), lambda i: (0,)),
        ],
        out_specs=pl.BlockSpec((BR, 2 * H), lambda i: (i, 0)),
        out_shape=jax.ShapeDtypeStruct((NE, 2 * H), jnp.float32),
    )(sums, w, b, ms)


def _gat_prep_body(x_ref, ab_ref, wg_ref, ms_ref, md_ref, xln_ref, hp_ref,
                   ad_ref, mxs_ref, mxd_ref):
    i = pl.program_id(0)
    xln = x_ref[:, :H] * ab_ref[:, :H] + ab_ref[:, H:]
    xln_ref[...] = xln
    h = jax.lax.dot_general(xln, wg_ref[...], (((1,), (1,)), ((), ())))
    asrcE = jnp.dot(h, ms_ref[...], preferred_element_type=jnp.float32)
    adstE = jnp.dot(h, md_ref[...], preferred_element_type=jnp.float32)
    hp_ref[...] = jnp.concatenate([h, asrcE], axis=1)
    ad_ref[...] = adstE
    bs = jnp.max(asrcE)
    bd = jnp.max(adstE)

    @pl.when(i == 0)
    def _():
        mxs_ref[0, 0] = bs
        mxd_ref[0, 0] = bd

    @pl.when(i > 0)
    def _():
        mxs_ref[0, 0] = jnp.maximum(mxs_ref[0, 0], bs)
        mxd_ref[0, 0] = jnp.maximum(mxd_ref[0, 0], bd)


def _gat_prep(xl1pack, ab_r, gat_w, A_s, A_d):
    BR = 2000
    return pl.pallas_call(
        _gat_prep_body,
        grid=(NL // BR,),
        in_specs=[
            pl.BlockSpec((BR, 2 * H + 8), lambda i: (i, 0)),
            pl.BlockSpec((BR, 2 * H), lambda i: (i, 0)),
            pl.BlockSpec((H, H), lambda i: (0, 0)),
            pl.BlockSpec((H, H), lambda i: (0, 0)),
            pl.BlockSpec((H, H), lambda i: (0, 0)),
        ],
        out_specs=(
            pl.BlockSpec((BR, H), lambda i: (i, 0)),
            pl.BlockSpec((BR, 2 * H), lambda i: (i, 0)),
            pl.BlockSpec((BR, H), lambda i: (i, 0)),
            pl.BlockSpec(memory_space=pltpu.SMEM),
            pl.BlockSpec(memory_space=pltpu.SMEM),
        ),
        out_shape=(
            jax.ShapeDtypeStruct((NL, H), jnp.float32),
            jax.ShapeDtypeStruct((NL, 2 * H), jnp.float32),
            jax.ShapeDtypeStruct((NL, H), jnp.float32),
            jax.ShapeDtypeStruct((1, 1), jnp.float32),
            jax.ShapeDtypeStruct((1, 1), jnp.float32),
        ),
    )(xl1pack, ab_r, gat_w, A_s, A_d)


def _head_expander():
    row = jax.lax.broadcasted_iota(jnp.int32, (16, H), 0)
    col = jax.lax.broadcasted_iota(jnp.int32, (16, H), 1)
    return (row == col // DH).astype(jnp.float32)


def _lane_picker():
    row = jax.lax.broadcasted_iota(jnp.int32, (H, 16), 0)
    col = jax.lax.broadcasted_iota(jnp.int32, (H, 16), 1)
    return ((col < HEADS) & (row == col * DH)).astype(jnp.float32)


def _gat_edge_body(gs_ref, gd_ref, c_ref, o_ref):
    s = gs_ref[:, H:] + gd_ref[...]
    alpha = jnp.where(s >= 0.0, s, 0.2 * s)
    ex128 = jnp.exp(alpha - c_ref[0, 0])
    ex16 = jnp.dot(ex128, _lane_picker(),
                   preferred_element_type=jnp.float32)
    o_ref[...] = jnp.concatenate([gs_ref[:, :H] * ex128, ex16], axis=1)


def _gat_edge(g_src, g_dst, cmax):
    BR = 4000
    NEL = g_src.shape[0]
    return pl.pallas_call(
        _gat_edge_body,
        grid=(NEL // BR,),
        in_specs=[
            pl.BlockSpec((BR, 2 * H), lambda i: (i, 0)),
            pl.BlockSpec((BR, H), lambda i: (i, 0)),
            pl.BlockSpec(memory_space=pltpu.SMEM),
        ],
        out_specs=pl.BlockSpec((BR, H + 16), lambda i: (i, 0)),
        out_shape=jax.ShapeDtypeStruct((NEL, H + 16), jnp.float32),
    )(g_src, g_dst, cmax)


def _gat_fin_body(nd_ref, xln_ref, b_ref, o_ref):
    den128 = jnp.dot(nd_ref[:, H:], _head_expander(),
                     preferred_element_type=jnp.float32)
    y = nd_ref[:, :H] / jnp.maximum(den128, 1e-16)
    xl2 = jnp.tanh(xln_ref[...] + y + b_ref[...][None, :])
    o_ref[...] = jnp.concatenate([xl2, xl2 * xl2], axis=1)


def _gat_fin(nd, xln, gat_b):
    BR = 2000
    return pl.pallas_call(
        _gat_fin_body,
        grid=(NL // BR,),
        in_specs=[
            pl.BlockSpec((BR, H + 16), lambda i: (i, 0)),
            pl.BlockSpec((BR, H), lambda i: (i, 0)),
            pl.BlockSpec((H,), lambda i: (0,)),
        ],
        out_specs=pl.BlockSpec((BR, 2 * H), lambda i: (i, 0)),
        out_shape=jax.ShapeDtypeStruct((NL, 2 * H), jnp.float32),
    )(nd, xln, gat_b)


def _score_tail_body(gT_ref, gH_ref, lT_ref, lH_ref, cT_ref, cH_ref,
                     gSw_ref, lSw_ref, gDw_ref, lDw_ref, gSb_ref, lSb_ref,
                     se_ref, de_ref):
    cT = cT_ref[...]
    cH = cH_ref[...]
    icT = 1.0 / jnp.maximum(cT, 1.0)
    icH = 1.0 / jnp.maximum(cH, 1.0)
    mskT = (cT > 0.0).astype(jnp.float32)
    mskH = (cH > 0.0).astype(jnp.float32)

    def one(sums_T, sums_H, Sw, Sb, Dw):
        MT = sums_T[:, :H] * icT
        MT2 = sums_T[:, H:2 * H] * icT
        MH = sums_H[:, :H] * icH
        MH2 = sums_H[:, H:2 * H] * icH
        d_T = MT2 - MT * MT
        d_H = MH2 - MH * MH
        d_TH = (MT2 - 2.0 * MT * MH + MH * MH) * mskT
        d_HT = (MH2 - 2.0 * MH * MT + MT * MT) * mskH
        lin = (
            jnp.sum(d_T * Sw[0, :H][None, :], axis=1, keepdims=True)
            + jnp.sum(d_H * Sw[0, H:2 * H][None, :], axis=1, keepdims=True)
            + jnp.sum(d_TH * Sw[0, 2 * H:3 * H][None, :], axis=1, keepdims=True)
            + jnp.sum(d_HT * Sw[0, 3 * H:][None, :], axis=1, keepdims=True)
        )
        se = jax.nn.sigmoid(lin + Sb)
        a = jax.lax.dot_general(MT, Dw, (((1,), (1,)), ((), ())))
        na = jnp.maximum(
            jnp.sqrt(jnp.sum(a * a, axis=1, keepdims=True)), 1e-8)
        nb = jnp.maximum(
            jnp.sqrt(jnp.sum(MH * MH, axis=1, keepdims=True)), 1e-8)
        de = (jnp.sum(a * MH, axis=1, keepdims=True) / (na * nb) + 1.0) / 2.0
        return se, de

    se_g, de_g = one(gT_ref[...], gH_ref[...], gSw_ref[...], gSb_ref[0, 0],
                     gDw_ref[...])
    se_l, de_l = one(lT_ref[...], lH_ref[...], lSw_ref[...], lSb_ref[0, 0],
                     lDw_ref[...])
    se_ref[...] = Q * se_g + (1.0 - Q) * se_l
    de_ref[...] = Q * de_g + (1.0 - Q) * de_l


def _score_tail(sums_gT, sums_gH, sums_lT, sums_lH, cT, cH,
                gSw, gSb, lSw, lSb, gDw, lDw):
    BR = 2000
    sspec = pl.BlockSpec((BR, 2 * H), lambda i: (i, 0))
    cspec = pl.BlockSpec((BR, 1), lambda i: (i, 0))
    wspec = pl.BlockSpec((1, 4 * H), lambda i: (0, 0))
    dspec = pl.BlockSpec((H, H), lambda i: (0, 0))
    return pl.pallas_call(
        _score_tail_body,
        grid=(NE // BR,),
        in_specs=[sspec, sspec, sspec, sspec, cspec, cspec,
                  wspec, wspec, dspec, dspec,
                  pl.BlockSpec(memory_space=pltpu.SMEM),
                  pl.BlockSpec(memory_space=pltpu.SMEM)],
        out_specs=(cspec, cspec),
        out_shape=(
            jax.ShapeDtypeStruct((NE, 1), jnp.float32),
            jax.ShapeDtypeStruct((NE, 1), jnp.float32),
        ),
    )(sums_gT, sums_gH, sums_lT, sums_lH, cT, cH, gSw, lSw, gDw, lDw,
      gSb.reshape(1, 1), lSb.reshape(1, 1))


def _mlp_body(x_ref, w1_ref, b1_ref, g_ref, b_ref, w2_ref, b2_ref, o_ref):
    hm = jax.nn.relu(
        jax.lax.dot_general(x_ref[...], w1_ref[...], (((1,), (1,)), ((), ())))
        + b1_ref[...][None, :]
    )
    mean = jnp.mean(hm, axis=1, keepdims=True)
    var = jnp.mean((hm - mean) ** 2, axis=1, keepdims=True)
    hm = (hm - mean) * jax.lax.rsqrt(var + 1e-5) * g_ref[...][None, :] + b_ref[...][
        None, :
    ]
    s = jnp.sum(hm * w2_ref[...], axis=1, keepdims=True) + b2_ref[0, 0]
    o_ref[...] = jax.nn.sigmoid(s)


def _edge_mlp(Xe, w1, b1, lng, lnb, w2, b2):
    BR = 2000
    return pl.pallas_call(
        _mlp_body,
        grid=(NE // BR,),
        in_specs=[
            pl.BlockSpec((BR, DG), lambda i: (i, 0)),
            pl.BlockSpec((HID, DG), lambda i: (0, 0)),
            pl.BlockSpec((HID,), lambda i: (0,)),
            pl.BlockSpec((HID,), lambda i: (0,)),
            pl.BlockSpec((HID,), lambda i: (0,)),
            pl.BlockSpec((1, HID), lambda i: (0, 0)),
            pl.BlockSpec(memory_space=pltpu.SMEM),
        ],
        out_specs=pl.BlockSpec((BR, 1), lambda i: (i, 0)),
        out_shape=jax.ShapeDtypeStruct((NE, 1), jnp.float32),
    )(Xe, w1, b1, lng, lnb, w2, b2.reshape(1, 1))


def _seg_sum(vals, ids, num):
    return jax.ops.segment_sum(vals, ids, num_segments=num)


def kernel(Xl, Xg, Xe, C_vertex, C_edge, T_vertex, H_vertex, T_edge, H_edge,
           e_index, ling_w, ling_b, linl_w, linl_b, mha_wi, mha_bi, mha_wo,
           mha_bo, normg_g, normg_b, norml_w, norml_b, norml_ms, gat_w,
           gat_att_src, gat_att_dst, gat_b, lingS_w, lingS_b, linlS_w,
           linlS_b, lingD_w, linlD_w, mlp_w1, mlp_b1, mlp_lng, mlp_lnb,
           mlp_w2, mlp_b2):
    f32 = jnp.float32

    xg2pack = _xg_path(Xg, ling_w, ling_b, mha_wi, mha_bi, mha_wo, mha_bo,
                       normg_g, normg_b)

    CT = C_vertex[T_vertex]
    CH = C_vertex[H_vertex]
    gvT = _sc_gather(xg2pack, CT)
    gvH = _sc_gather(xg2pack, CH)
    sums_gT = _seg_sum(gvT, T_edge, NE)
    sums_gH = _seg_sum(gvH, H_edge, NE)
    onesTH = jnp.concatenate([
        jnp.tile(jnp.array([[1.0, 0.0]], f32), (NT, 1)),
        jnp.tile(jnp.array([[0.0, 1.0]], f32), (NH, 1)),
    ])
    cTH = _seg_sum(onesTH, jnp.concatenate([T_edge, H_edge]), NE)
    cT = cTH[:, 0:1]
    cH = cTH[:, 1:2]

    xl1pack = _xl_linear(Xl, linl_w, linl_b)
    gn_sums = _seg_sum(xl1pack, C_edge, NE)
    ab = _gn_stats(gn_sums, norml_w, norml_b, norml_ms)
    ab_r = _sc_gather(ab, C_edge)

    lane = jnp.arange(H, dtype=jnp.int32)
    head = lane // DH
    onehot = (head[:, None] == jnp.arange(HEADS, dtype=jnp.int32)[None, :]
              ).astype(f32)
    expand = (jnp.arange(HEADS, dtype=jnp.int32)[:, None] == head[None, :]
              ).astype(f32)
    M_s = (gat_att_src.reshape(H)[:, None] * onehot) @ expand
    M_d = (gat_att_dst.reshape(H)[:, None] * onehot) @ expand

    xln, hpack, adstE, mxs, mxd = _gat_prep(xl1pack, ab_r, gat_w, M_s, M_d)
    amax = mxs[0, 0] + mxd[0, 0]
    cmax = jnp.where(amax >= 0.0, amax, 0.2 * amax).reshape(1, 1)

    src = e_index[0]
    dst = e_index[1]
    g_src = _sc_gather(hpack, src)
    g_dst = _sc_gather(adstE, dst)
    edge_out = _gat_edge(g_src, g_dst, cmax)
    nd = _seg_sum(edge_out, dst, NL)
    xl2pack = _gat_fin(nd, xln, gat_b)

    lvT = _sc_gather(xl2pack, T_vertex)
    lvH = _sc_gather(xl2pack, H_vertex)
    sums_lT = _seg_sum(lvT, T_edge, NE)
    sums_lH = _seg_sum(lvH, H_edge, NE)

    Se, De = _score_tail(sums_gT, sums_gH, sums_lT, sums_lH, cT, cH,
                         lingS_w, lingS_b, linlS_w, linlS_b, lingD_w, linlD_w)

    Pe = _edge_mlp(Xe, mlp_w1, mlp_b1, mlp_lng, mlp_lnb, mlp_w2, mlp_b2)
    return Pe, Se, De


# speedup vs baseline: 14.8830x; 1.0774x over previous
import functools

import jax
import jax.numpy as jnp
import numpy as np
from jax import lax
from jax.experimental import pallas as pl
from jax.experimental.pallas import tpu as pltpu
from jax.experimental.pallas import tpu_sc as plsc

NG = 2048
NL = 200000
NE = 20000
NT = 100000
NH = 100000
EL = 600000
DG = 128
DL = 128
H = 128
HEADS = 8
DH = H // HEADS
HID = 128
Q = 0.5

_NC = 2
_NS = 16
_NW = _NC * _NS


def _sc_gather_call(table, idx, chunk):
    V, W = table.shape
    B = idx.shape[0]
    per_w = B // _NW
    n_chunks = per_w // chunk
    mesh = plsc.VectorSubcoreMesh(core_axis_name="c", subcore_axis_name="s")

    @functools.partial(
        pl.kernel,
        mesh=mesh,
        out_type=jax.ShapeDtypeStruct((B, W), jnp.float32),
        scratch_types=[
            pltpu.VMEM((chunk,), jnp.int32),
            pltpu.VMEM((chunk, W), jnp.float32),
            pltpu.SemaphoreType.DMA,
        ],
    )
    def k(table_hbm, idx_hbm, out_hbm, idx_v, rows_v, sem):
        wid = lax.axis_index("s") * _NC + lax.axis_index("c")
        base = wid * per_w

        def body(ci, _):
            off = base + ci * chunk
            pltpu.sync_copy(idx_hbm.at[pl.ds(off, chunk)], idx_v)
            pltpu.async_copy(table_hbm.at[idx_v], rows_v, sem).wait()
            pltpu.sync_copy(rows_v, out_hbm.at[pl.ds(off, chunk)])
            return ()

        lax.fori_loop(0, n_chunks, body, ())

    return k(table, idx)


def _pad_idx(idx, granule):
    n = idx.shape[0]
    m = ((n + granule - 1) // granule) * granule
    if m == n:
        return idx, n
    return jnp.concatenate([idx, jnp.zeros((m - n,), jnp.int32)]), n


def _sc_gather(table, idx):
    W = table.shape[1]
    chunk = 512 if W <= 128 else 256
    pidx, _ = _pad_idx(idx, chunk * _NW)
    return _sc_gather_call(table, pidx, chunk)


def _xg0_qkv_body(xg_ref, lw_ref, lb_ref, wi_ref, bi_ref, xg0_ref, qkv_ref):
    x = xg_ref[...]
    xg0 = jnp.tanh(
        jax.lax.dot_general(x, lw_ref[...], (((1,), (1,)), ((), ())))
        + lb_ref[...][None, :]
    )
    xg0_ref[...] = xg0
    qkv_ref[...] = (
        jax.lax.dot_general(xg0, wi_ref[...], (((1,), (1,)), ((), ())))
        + bi_ref[...][None, :]
    )


def _attn_body(q_ref, k_ref, v_ref, o_ref):
    q = q_ref[0]
    k = k_ref[0]
    v = v_ref[0]
    s = jax.lax.dot_general(q, k, (((1,), (1,)), ((), ()))) * (
        1.0 / np.sqrt(DH).astype(np.float32)
    )
    m = jnp.max(s, axis=1, keepdims=True)
    e = jnp.exp(s - m)
    p = e / jnp.sum(e, axis=1, keepdims=True)
    o_ref[0] = jnp.dot(p, v, preferred_element_type=jnp.float32)


def _xg_out_body(xg0_ref, at_ref, wo_ref, bo_ref, g_ref, b_ref, out_ref):
    xg0 = xg0_ref[...]
    xg1 = (
        jax.lax.dot_general(at_ref[...], wo_ref[...], (((1,), (1,)), ((), ())))
        + bo_ref[...][None, :]
    )
    y = xg0 + xg1
    mean = jnp.mean(y, axis=1, keepdims=True)
    var = jnp.mean((y - mean) ** 2, axis=1, keepdims=True)
    xg2 = jnp.tanh(
        (y - mean) * jax.lax.rsqrt(var + 1e-5) * g_ref[...][None, :]
        + b_ref[...][None, :]
    )
    out_ref[...] = jnp.concatenate([xg2, xg2 * xg2], axis=1)


def _xg_path(Xg, ling_w, ling_b, mha_wi, mha_bi, mha_wo, mha_bo, normg_g, normg_b):
    xg0, qkv = pl.pallas_call(
        _xg0_qkv_body,
        out_shape=(
            jax.ShapeDtypeStruct((NG, H), jnp.float32),
            jax.ShapeDtypeStruct((NG, 3 * H), jnp.float32),
        ),
    )(Xg, ling_w, ling_b, mha_wi, mha_bi)

    qkvh = qkv.reshape(NG, 3 * HEADS, DH).transpose(1, 0, 2)
    BQ = 256
    NQB = NG // BQ
    attnh = pl.pallas_call(
        _attn_body,
        grid=(HEADS, NQB),
        in_specs=[
            pl.BlockSpec((1, BQ, DH), lambda h, qb: (h, qb, 0)),
            pl.BlockSpec((1, NG, DH), lambda h, qb: (HEADS + h, 0, 0)),
            pl.BlockSpec((1, NG, DH), lambda h, qb: (2 * HEADS + h, 0, 0)),
        ],
        out_specs=pl.BlockSpec((1, BQ, DH), lambda h, qb: (h, qb, 0)),
        out_shape=jax.ShapeDtypeStruct((HEADS, NG, DH), jnp.float32),
    )(qkvh, qkvh, qkvh)
    attn = attnh.transpose(1, 0, 2).reshape(NG, H)

    xg2pack = pl.pallas_call(
        _xg_out_body,
        out_shape=jax.ShapeDtypeStruct((NG, 2 * H), jnp.float32),
    )(xg0, attn, mha_wo, mha_bo, normg_g, normg_b)
    return xg2pack


def _linear_tanh_sq_body(x_ref, w_ref, b_ref, o_ref):
    y = jnp.tanh(
        jax.lax.dot_general(x_ref[...], w_ref[...], (((1,), (1,)), ((), ())))
        + b_ref[...][None, :]
    )
    ones = jnp.ones((y.shape[0], 8), jnp.float32)
    o_ref[...] = jnp.concatenate([y, y * y, ones], axis=1)


def _xl_linear(Xl, linl_w, linl_b):
    BR = 2000
    return pl.pallas_call(
        _linear_tanh_sq_body,
        grid=(NL // BR,),
        in_specs=[
            pl.BlockSpec((BR, DL), lambda i: (i, 0)),
            pl.BlockSpec((H, DL), lambda i: (0, 0)),
            pl.BlockSpec((H,), lambda i: (0,)),
        ],
        out_specs=pl.BlockSpec((BR, 2 * H + 8), lambda i: (i, 0)),
        out_shape=jax.ShapeDtypeStruct((NL, 2 * H + 8), jnp.float32),
    )(Xl, linl_w, linl_b)


def _gn_stats_body(s_ref, w_ref, b_ref, ms_ref, ab_ref):
    c = jnp.maximum(s_ref[:, 2 * H:2 * H + 1], 1.0)
    ic = 1.0 / c
    mean = s_ref[:, :H] * ic
    msq = s_ref[:, H:2 * H] * ic
    cvec = mean * ms_ref[...][None, :]
    var = msq - 2.0 * cvec * mean + cvec * cvec
    A = w_ref[...][None, :] * jax.lax.rsqrt(var + 1e-5)
    B = b_ref[...][None, :] - cvec * A
    ab_ref[...] = jnp.concatenate([A, B], axis=1)


def _gn_stats(sums, w, b, ms):
    BR = 2000
    return pl.pallas_call(
        _gn_stats_body,
        grid=(NE // BR,),
        in_specs=[
            pl.BlockSpec((BR, 2 * H + 8), lambda i: (i, 0)),
            pl.BlockSpec((H,), lambda i: (0,)),
            pl.BlockSpec((H,), lambda i: (0,)),
            pl.BlockSpec((H,), lambda i: (0,)),
        ],
        out_specs=pl.BlockSpec((BR, 2 * H), lambda i: (i, 0)),
        out_shape=jax.ShapeDtypeStruct((NE, 2 * H), jnp.float32),
    )(sums, w, b, ms)


def _gat_prep_body(x_ref, ab_ref, wg_ref, ms_ref, md_ref, xln_ref, hp_ref,
                   ad_ref, mxs_ref, mxd_ref):
    i = pl.program_id(0)
    xln = x_ref[:, :H] * ab_ref[:, :H] + ab_ref[:, H:]
    xln_ref[...] = xln
    h = jax.lax.dot_general(xln, wg_ref[...], (((1,), (1,)), ((), ())))
    asrcE = jnp.dot(h, ms_ref[...], preferred_element_type=jnp.float32)
    adstE = jnp.dot(h, md_ref[...], preferred_element_type=jnp.float32)
    hp_ref[...] = jnp.concatenate([h, asrcE], axis=1)
    ad_ref[...] = adstE
    bs = jnp.max(asrcE)
    bd = jnp.max(adstE)

    @pl.when(i == 0)
    def _():
        mxs_ref[0, 0] = bs
        mxd_ref[0, 0] = bd

    @pl.when(i > 0)
    def _():
        mxs_ref[0, 0] = jnp.maximum(mxs_ref[0, 0], bs)
        mxd_ref[0, 0] = jnp.maximum(mxd_ref[0, 0], bd)


def _gat_prep(xl1pack, ab_r, gat_w, A_s, A_d):
    BR = 2000
    return pl.pallas_call(
        _gat_prep_body,
        grid=(NL // BR,),
        in_specs=[
            pl.BlockSpec((BR, 2 * H + 8), lambda i: (i, 0)),
            pl.BlockSpec((BR, 2 * H), lambda i: (i, 0)),
            pl.BlockSpec((H, H), lambda i: (0, 0)),
            pl.BlockSpec((H, H), lambda i: (0, 0)),
            pl.BlockSpec((H, H), lambda i: (0, 0)),
        ],
        out_specs=(
            pl.BlockSpec((BR, H), lambda i: (i, 0)),
            pl.BlockSpec((BR, 2 * H), lambda i: (i, 0)),
            pl.BlockSpec((BR, H), lambda i: (i, 0)),
            pl.BlockSpec(memory_space=pltpu.SMEM),
            pl.BlockSpec(memory_space=pltpu.SMEM),
        ),
        out_shape=(
            jax.ShapeDtypeStruct((NL, H), jnp.float32),
            jax.ShapeDtypeStruct((NL, 2 * H), jnp.float32),
            jax.ShapeDtypeStruct((NL, H), jnp.float32),
            jax.ShapeDtypeStruct((1, 1), jnp.float32),
            jax.ShapeDtypeStruct((1, 1), jnp.float32),
        ),
    )(xl1pack, ab_r, gat_w, A_s, A_d)


def _head_expander():
    row = jax.lax.broadcasted_iota(jnp.int32, (16, H), 0)
    col = jax.lax.broadcasted_iota(jnp.int32, (16, H), 1)
    return (row == col // DH).astype(jnp.float32)


def _lane_picker():
    row = jax.lax.broadcasted_iota(jnp.int32, (H, 16), 0)
    col = jax.lax.broadcasted_iota(jnp.int32, (H, 16), 1)
    return ((col < HEADS) & (row == col * DH)).astype(jnp.float32)


def _gat_edge_body(gs_ref, gd_ref, c_ref, o_ref):
    i = pl.program_id(0)
    BR = gs_ref.shape[0]
    s = gs_ref[:, H:] + gd_ref[...]
    alpha = jnp.where(s >= 0.0, s, 0.2 * s)
    rid = i * BR + jax.lax.broadcasted_iota(jnp.int32, (BR, 1), 0)
    msk = (rid < EL).astype(jnp.float32)
    ex128 = jnp.exp(alpha - c_ref[0, 0]) * msk
    ex16 = jnp.dot(ex128, _lane_picker(),
                   preferred_element_type=jnp.float32)
    o_ref[...] = jnp.concatenate([gs_ref[:, :H] * ex128, ex16], axis=1)


def _gat_edge(g_src, g_dst, cmax):
    BR = 4096
    NEL = g_src.shape[0]
    return pl.pallas_call(
        _gat_edge_body,
        grid=(NEL // BR,),
        in_specs=[
            pl.BlockSpec((BR, 2 * H), lambda i: (i, 0)),
            pl.BlockSpec((BR, H), lambda i: (i, 0)),
            pl.BlockSpec(memory_space=pltpu.SMEM),
        ],
        out_specs=pl.BlockSpec((BR, H + 16), lambda i: (i, 0)),
        out_shape=jax.ShapeDtypeStruct((NEL, H + 16), jnp.float32),
    )(g_src, g_dst, cmax)


def _gat_fin_body(nd_ref, xln_ref, b_ref, o_ref):
    den128 = jnp.dot(nd_ref[:, H:], _head_expander(),
                     preferred_element_type=jnp.float32)
    y = nd_ref[:, :H] / jnp.maximum(den128, 1e-16)
    xl2 = jnp.tanh(xln_ref[...] + y + b_ref[...][None, :])
    o_ref[...] = jnp.concatenate([xl2, xl2 * xl2], axis=1)


def _gat_fin(nd, xln, gat_b):
    BR = 2000
    return pl.pallas_call(
        _gat_fin_body,
        grid=(NL // BR,),
        in_specs=[
            pl.BlockSpec((BR, H + 16), lambda i: (i, 0)),
            pl.BlockSpec((BR, H), lambda i: (i, 0)),
            pl.BlockSpec((H,), lambda i: (0,)),
        ],
        out_specs=pl.BlockSpec((BR, 2 * H), lambda i: (i, 0)),
        out_shape=jax.ShapeDtypeStruct((NL, 2 * H), jnp.float32),
    )(nd, xln, gat_b)


def _score_tail_body(gT_ref, gH_ref, lT_ref, lH_ref, cT_ref, cH_ref,
                     gSw_ref, lSw_ref, gDw_ref, lDw_ref, gSb_ref, lSb_ref,
                     se_ref, de_ref):
    cT = cT_ref[...]
    cH = cH_ref[...]
    icT = 1.0 / jnp.maximum(cT, 1.0)
    icH = 1.0 / jnp.maximum(cH, 1.0)
    mskT = (cT > 0.0).astype(jnp.float32)
    mskH = (cH > 0.0).astype(jnp.float32)

    def one(sums_T, sums_H, Sw, Sb, Dw):
        MT = sums_T[:, :H] * icT
        MT2 = sums_T[:, H:2 * H] * icT
        MH = sums_H[:, :H] * icH
        MH2 = sums_H[:, H:2 * H] * icH
        d_T = MT2 - MT * MT
        d_H = MH2 - MH * MH
        d_TH = (MT2 - 2.0 * MT * MH + MH * MH) * mskT
        d_HT = (MH2 - 2.0 * MH * MT + MT * MT) * mskH
        lin = (
            jnp.sum(d_T * Sw[0, :H][None, :], axis=1, keepdims=True)
            + jnp.sum(d_H * Sw[0, H:2 * H][None, :], axis=1, keepdims=True)
            + jnp.sum(d_TH * Sw[0, 2 * H:3 * H][None, :], axis=1, keepdims=True)
            + jnp.sum(d_HT * Sw[0, 3 * H:][None, :], axis=1, keepdims=True)
        )
        se = jax.nn.sigmoid(lin + Sb)
        a = jax.lax.dot_general(MT, Dw, (((1,), (1,)), ((), ())))
        na = jnp.maximum(
            jnp.sqrt(jnp.sum(a * a, axis=1, keepdims=True)), 1e-8)
        nb = jnp.maximum(
            jnp.sqrt(jnp.sum(MH * MH, axis=1, keepdims=True)), 1e-8)
        de = (jnp.sum(a * MH, axis=1, keepdims=True) / (na * nb) + 1.0) / 2.0
        return se, de

    se_g, de_g = one(gT_ref[...], gH_ref[...], gSw_ref[...], gSb_ref[0, 0],
                     gDw_ref[...])
    se_l, de_l = one(lT_ref[...], lH_ref[...], lSw_ref[...], lSb_ref[0, 0],
                     lDw_ref[...])
    se_ref[...] = Q * se_g + (1.0 - Q) * se_l
    de_ref[...] = Q * de_g + (1.0 - Q) * de_l


def _score_tail(sums_gT, sums_gH, sums_lT, sums_lH, cT, cH,
                gSw, gSb, lSw, lSb, gDw, lDw):
    BR = 2000
    sspec = pl.BlockSpec((BR, 2 * H), lambda i: (i, 0))
    cspec = pl.BlockSpec((BR, 1), lambda i: (i, 0))
    wspec = pl.BlockSpec((1, 4 * H), lambda i: (0, 0))
    dspec = pl.BlockSpec((H, H), lambda i: (0, 0))
    return pl.pallas_call(
        _score_tail_body,
        grid=(NE // BR,),
        in_specs=[sspec, sspec, sspec, sspec, cspec, cspec,
                  wspec, wspec, dspec, dspec,
                  pl.BlockSpec(memory_space=pltpu.SMEM),
                  pl.BlockSpec(memory_space=pltpu.SMEM)],
        out_specs=(cspec, cspec),
        out_shape=(
            jax.ShapeDtypeStruct((NE, 1), jnp.float32),
            jax.ShapeDtypeStruct((NE, 1), jnp.float32),
        ),
    )(sums_gT, sums_gH, sums_lT, sums_lH, cT, cH, gSw, lSw, gDw, lDw,
      gSb.reshape(1, 1), lSb.reshape(1, 1))


def _mlp_body(x_ref, w1_ref, b1_ref, g_ref, b_ref, w2_ref, b2_ref, o_ref):
    hm = jax.nn.relu(
        jax.lax.dot_general(x_ref[...], w1_ref[...], (((1,), (1,)), ((), ())))
        + b1_ref[...][None, :]
    )
    mean = jnp.mean(hm, axis=1, keepdims=True)
    var = jnp.mean((hm - mean) ** 2, axis=1, keepdims=True)
    hm = (hm - mean) * jax.lax.rsqrt(var + 1e-5) * g_ref[...][None, :] + b_ref[...][
        None, :
    ]
    s = jnp.sum(hm * w2_ref[...], axis=1, keepdims=True) + b2_ref[0, 0]
    o_ref[...] = jax.nn.sigmoid(s)


def _edge_mlp(Xe, w1, b1, lng, lnb, w2, b2):
    BR = 2000
    return pl.pallas_call(
        _mlp_body,
        grid=(NE // BR,),
        in_specs=[
            pl.BlockSpec((BR, DG), lambda i: (i, 0)),
            pl.BlockSpec((HID, DG), lambda i: (0, 0)),
            pl.BlockSpec((HID,), lambda i: (0,)),
            pl.BlockSpec((HID,), lambda i: (0,)),
            pl.BlockSpec((HID,), lambda i: (0,)),
            pl.BlockSpec((1, HID), lambda i: (0, 0)),
            pl.BlockSpec(memory_space=pltpu.SMEM),
        ],
        out_specs=pl.BlockSpec((BR, 1), lambda i: (i, 0)),
        out_shape=jax.ShapeDtypeStruct((NE, 1), jnp.float32),
    )(Xe, w1, b1, lng, lnb, w2, b2.reshape(1, 1))


def _seg_sum(vals, ids, num):
    return jax.ops.segment_sum(vals, ids, num_segments=num)


def kernel(Xl, Xg, Xe, C_vertex, C_edge, T_vertex, H_vertex, T_edge, H_edge,
           e_index, ling_w, ling_b, linl_w, linl_b, mha_wi, mha_bi, mha_wo,
           mha_bo, normg_g, normg_b, norml_w, norml_b, norml_ms, gat_w,
           gat_att_src, gat_att_dst, gat_b, lingS_w, lingS_b, linlS_w,
           linlS_b, lingD_w, linlD_w, mlp_w1, mlp_b1, mlp_lng, mlp_lnb,
           mlp_w2, mlp_b2):
    f32 = jnp.float32

    xg2pack = _xg_path(Xg, ling_w, ling_b, mha_wi, mha_bi, mha_wo, mha_bo,
                       normg_g, normg_b)

    CT = C_vertex[T_vertex]
    CH = C_vertex[H_vertex]
    gvT = _sc_gather(xg2pack, CT)
    gvH = _sc_gather(xg2pack, CH)
    padT = gvT.shape[0] - NT
    T_edge_p = jnp.concatenate([T_edge, jnp.full((padT,), NE, jnp.int32)])
    H_edge_p = jnp.concatenate([H_edge, jnp.full((padT,), NE, jnp.int32)])
    sums_gT = _seg_sum(gvT, T_edge_p, NE + 1)
    sums_gH = _seg_sum(gvH, H_edge_p, NE + 1)
    onesTH = jnp.concatenate([
        jnp.tile(jnp.array([[1.0, 0.0]], f32), (NT, 1)),
        jnp.tile(jnp.array([[0.0, 1.0]], f32), (NH, 1)),
    ])
    cTH = _seg_sum(onesTH, jnp.concatenate([T_edge, H_edge]), NE)
    cT = cTH[:, 0:1]
    cH = cTH[:, 1:2]

    xl1pack = _xl_linear(Xl, linl_w, linl_b)
    gn_sums = _seg_sum(xl1pack, C_edge, NE)
    ab = _gn_stats(gn_sums, norml_w, norml_b, norml_ms)
    ab_r = _sc_gather(ab, C_edge)

    lane = jnp.arange(H, dtype=jnp.int32)
    head = lane // DH
    onehot = (head[:, None] == jnp.arange(HEADS, dtype=jnp.int32)[None, :]
              ).astype(f32)
    expand = (jnp.arange(HEADS, dtype=jnp.int32)[:, None] == head[None, :]
              ).astype(f32)
    M_s = (gat_att_src.reshape(H)[:, None] * onehot) @ expand
    M_d = (gat_att_dst.reshape(H)[:, None] * onehot) @ expand

    xln, hpack, adstE, mxs, mxd = _gat_prep(xl1pack, ab_r, gat_w, M_s, M_d)
    amax = mxs[0, 0] + mxd[0, 0]
    cmax = jnp.where(amax >= 0.0, amax, 0.2 * amax).reshape(1, 1)

    P = ((EL + 16383) // 16384) * 16384
    zpad = jnp.zeros((P - EL,), jnp.int32)
    srcp = jnp.concatenate([e_index[0], zpad])
    dstp = jnp.concatenate([e_index[1], zpad])
    g_src = _sc_gather(hpack, srcp)
    g_dst = _sc_gather(adstE, dstp)
    edge_out = _gat_edge(g_src, g_dst, cmax)
    nd = _seg_sum(edge_out, dstp, NL)
    xl2pack = _gat_fin(nd, xln, gat_b)

    lvT = _sc_gather(xl2pack, T_vertex)
    lvH = _sc_gather(xl2pack, H_vertex)
    sums_lT = _seg_sum(lvT, T_edge_p, NE + 1)
    sums_lH = _seg_sum(lvH, H_edge_p, NE + 1)

    Se, De = _score_tail(sums_gT, sums_gH, sums_lT, sums_lH, cT, cH,
                         lingS_w, lingS_b, linlS_w, linlS_b, lingD_w, linlD_w)

    Pe = _edge_mlp(Xe, mlp_w1, mlp_b1, mlp_lng, mlp_lnb, mlp_w2, mlp_b2)
    return Pe, Se, De
